# baseline (speedup 1.0000x reference)
import functools

import jax
import jax.numpy as jnp
from jax import lax
from jax.experimental import pallas as pl
from jax.experimental.pallas import tpu as pltpu
from jax.experimental.pallas import tpu_sc as plsc

VOCAB_ROWS = 202
EMBED = 32
BLK_B = 128
NBUF = 2
TILE = 8 * BLK_B


def _make_lookup(n_b, n_h):
    info = plsc.get_sparse_core_info()
    nc, ns = info.num_cores, info.num_subcores
    nw = nc * ns
    b_per_w = n_b // nw
    blocks = b_per_w // BLK_B
    dt_n = EMBED // 8
    n_bt = n_b // BLK_B

    mesh = plsc.VectorSubcoreMesh(core_axis_name="c", subcore_axis_name="s")

    @functools.partial(
        pl.kernel,
        mesh=mesh,
        compiler_params=pltpu.CompilerParams(
            use_tc_tiling_on_sc=False, needs_layout_passes=False
        ),
        out_type=jax.ShapeDtypeStruct((n_h, dt_n, n_bt * TILE), jnp.float32),
        scratch_types=[
            pltpu.VMEM((EMBED, VOCAB_ROWS), jnp.float32),
            pltpu.VMEM((BLK_B, n_h + 1), jnp.int32),
            pltpu.VMEM((NBUF, dt_n * TILE), jnp.float32),
            pltpu.SemaphoreType.DMA((NBUF,)),
        ],
    )
    def k(table_hbm, loc_hbm, out_hbm, table_v, loc_v, out_v, sem_out):
        wid = lax.axis_index("s") * nc + lax.axis_index("c")
        pltpu.sync_copy(table_hbm, table_v)
        iota16 = lax.iota(jnp.int32, 16)

        def wait_out(buf):
            pltpu.make_async_copy(
                out_v.at[buf], out_hbm.at[0, 0, pl.ds(0, dt_n * TILE)],
                sem_out.at[buf],
            ).wait()

        def process_h(h, bt, buf):
            idxs = [
                plsc.load_gather(
                    loc_v, [iota16 + (j * 16), jnp.full((16,), h, jnp.int32)]
                )
                for j in range(BLK_B // 16)
            ]

            @plsc.parallel_loop(0, EMBED, unroll=4)
            def _(d):
                dsplat = jnp.full((16,), 1, jnp.int32) * d
                for j in range(BLK_B // 16):
                    v = plsc.load_gather(table_v, [dsplat, idxs[j]])
                    out_v[buf, pl.ds(d * BLK_B + j * 16, 16)] = v

            for dt in range(dt_n):
                pltpu.async_copy(
                    out_v.at[buf, pl.ds(dt * TILE, TILE)],
                    out_hbm.at[h, dt, pl.ds(bt * TILE, TILE)],
                    sem_out.at[buf],
                )

        @pl.loop(0, blocks)
        def _(blk_i):
            bt = wid * blocks + blk_i
            pltpu.sync_copy(
                loc_hbm.at[pl.ds(bt * BLK_B, BLK_B), :],
                loc_v.at[:, pl.ds(0, n_h)],
            )

            @pl.loop(0, n_h // NBUF)
            def _(i):
                h0 = i * NBUF
                not_first = (blk_i + i) > 0
                pl.when(not_first)(lambda: wait_out(0))
                process_h(h0, bt, 0)
                pl.when(not_first)(lambda: wait_out(1))
                process_h(h0 + 1, bt, 1)

        wait_out(0)
        wait_out(1)

    return k


def kernel(location, table):
    b, h = location.shape
    q = _make_lookup(b, h)(table.T, location.astype(jnp.int32))
    q5 = q.reshape(h, EMBED // 8, b // BLK_B, 8, BLK_B)
    return q5.transpose(2, 4, 0, 1, 3).reshape(b, h, EMBED)


# speedup vs baseline: 1.0341x; 1.0341x over previous
import functools

import jax
import jax.numpy as jnp
from jax import lax
from jax.experimental import pallas as pl
from jax.experimental.pallas import tpu as pltpu
from jax.experimental.pallas import tpu_sc as plsc

VOCAB_ROWS = 202
EMBED = 32
BLK_B = 128
NBUF = 2
TILE = 8 * BLK_B


def _make_lookup(n_b, n_h):
    info = plsc.get_sparse_core_info()
    nc, ns = info.num_cores, info.num_subcores
    nw = nc * ns
    b_per_w = n_b // nw
    blocks = b_per_w // BLK_B
    dt_n = EMBED // 8
    n_bt = n_b // BLK_B

    mesh = plsc.VectorSubcoreMesh(core_axis_name="c", subcore_axis_name="s")

    @functools.partial(
        pl.kernel,
        mesh=mesh,
        compiler_params=pltpu.CompilerParams(
            use_tc_tiling_on_sc=False, needs_layout_passes=False
        ),
        out_type=jax.ShapeDtypeStruct((n_h, dt_n, n_bt * TILE), jnp.float32),
        scratch_types=[
            pltpu.VMEM((EMBED, VOCAB_ROWS), jnp.float32),
            pltpu.VMEM((BLK_B, n_h), jnp.int32),
            pltpu.VMEM((NBUF, dt_n * TILE), jnp.float32),
            pltpu.SemaphoreType.DMA((NBUF,)),
        ],
    )
    def k(table_hbm, loc_hbm, out_hbm, table_v, loc_v, out_v, sem_out):
        wid = lax.axis_index("s") * nc + lax.axis_index("c")
        pltpu.sync_copy(table_hbm, table_v)
        iota16 = lax.iota(jnp.int32, 16)

        def wait_out(buf):
            pltpu.make_async_copy(
                out_v.at[buf], out_hbm.at[0, 0, pl.ds(0, dt_n * TILE)],
                sem_out.at[buf],
            ).wait()

        def process_h(h, bt, buf):
            idxs = [
                plsc.load_gather(
                    loc_v, [iota16 + (j * 16), jnp.full((16,), h, jnp.int32)]
                )
                for j in range(BLK_B // 16)
            ]

            @plsc.parallel_loop(0, EMBED, unroll=8)
            def _(d):
                dsplat = jnp.full((16,), 1, jnp.int32) * d
                for j in range(BLK_B // 16):
                    v = plsc.load_gather(table_v, [dsplat, idxs[j]])
                    out_v[buf, pl.ds(d * BLK_B + j * 16, 16)] = v

            for dt in range(dt_n):
                pltpu.async_copy(
                    out_v.at[buf, pl.ds(dt * TILE, TILE)],
                    out_hbm.at[h, dt, pl.ds(bt * TILE, TILE)],
                    sem_out.at[buf],
                )

        @pl.loop(0, blocks)
        def _(blk_i):
            bt = wid * blocks + blk_i
            pltpu.sync_copy(loc_hbm.at[pl.ds(bt * BLK_B, BLK_B), :], loc_v)

            @pl.loop(0, n_h // NBUF)
            def _(i):
                h0 = i * NBUF
                not_first = (blk_i + i) > 0
                pl.when(not_first)(lambda: wait_out(0))
                process_h(h0, bt, 0)
                pl.when(not_first)(lambda: wait_out(1))
                process_h(h0 + 1, bt, 1)

        wait_out(0)
        wait_out(1)

    return k


def kernel(location, table):
    b, h = location.shape
    q = _make_lookup(b, h)(table.T, location.astype(jnp.int32))
    q5 = q.reshape(h, EMBED // 8, b // BLK_B, 8, BLK_B)
    return q5.transpose(2, 4, 0, 1, 3).reshape(b, h, EMBED)


# speedup vs baseline: 1.0657x; 1.0305x over previous
import functools

import jax
import jax.numpy as jnp
from jax import lax
from jax.experimental import pallas as pl
from jax.experimental.pallas import tpu as pltpu
from jax.experimental.pallas import tpu_sc as plsc

VOCAB_ROWS = 202
EMBED = 32
BLK_B = 128
NBUF = 2
TILE = 8 * BLK_B


def _make_lookup(n_b, n_h):
    info = plsc.get_sparse_core_info()
    nc, ns = info.num_cores, info.num_subcores
    nw = nc * ns
    b_per_w = n_b // nw
    blocks = b_per_w // BLK_B
    dt_n = EMBED // 8
    n_bt = n_b // BLK_B

    mesh = plsc.VectorSubcoreMesh(core_axis_name="c", subcore_axis_name="s")

    @functools.partial(
        pl.kernel,
        mesh=mesh,
        compiler_params=pltpu.CompilerParams(
            use_tc_tiling_on_sc=False, needs_layout_passes=False
        ),
        out_type=jax.ShapeDtypeStruct((n_h, dt_n, n_bt * TILE), jnp.float32),
        scratch_types=[
            pltpu.VMEM((EMBED, VOCAB_ROWS), jnp.float32),
            pltpu.VMEM((2, BLK_B, n_h), jnp.int32),
            pltpu.VMEM((NBUF, dt_n * TILE), jnp.float32),
            pltpu.SemaphoreType.DMA((NBUF,)),
            pltpu.SemaphoreType.DMA((2,)),
        ],
    )
    def k(table_hbm, loc_hbm, out_hbm, table_v, loc_v, out_v, sem_out,
          sem_loc):
        wid = lax.axis_index("s") * nc + lax.axis_index("c")
        pltpu.sync_copy(table_hbm, table_v)
        iota16 = lax.iota(jnp.int32, 16)

        def start_loc(blk_i, lb):
            bt = wid * blocks + blk_i
            pltpu.async_copy(
                loc_hbm.at[pl.ds(bt * BLK_B, BLK_B), :], loc_v.at[lb],
                sem_loc.at[lb],
            )

        def wait_loc(lb):
            pltpu.make_async_copy(
                loc_hbm.at[pl.ds(0, BLK_B), :], loc_v.at[lb], sem_loc.at[lb]
            ).wait()

        def wait_out(buf):
            pltpu.make_async_copy(
                out_v.at[buf], out_hbm.at[0, 0, pl.ds(0, dt_n * TILE)],
                sem_out.at[buf],
            ).wait()

        def process_h(h, bt, buf, lb):
            loc_ref = loc_v.at[lb]
            idxs = [
                plsc.load_gather(
                    loc_ref,
                    [iota16 + (j * 16), jnp.full((16,), h, jnp.int32)],
                )
                for j in range(BLK_B // 16)
            ]

            @plsc.parallel_loop(0, EMBED, unroll=4)
            def _(d):
                dsplat = jnp.full((16,), 1, jnp.int32) * d
                for j in range(BLK_B // 16):
                    v = plsc.load_gather(table_v, [dsplat, idxs[j]])
                    out_v[buf, pl.ds(d * BLK_B + j * 16, 16)] = v

            for dt in range(dt_n):
                pltpu.async_copy(
                    out_v.at[buf, pl.ds(dt * TILE, TILE)],
                    out_hbm.at[h, dt, pl.ds(bt * TILE, TILE)],
                    sem_out.at[buf],
                )

        start_loc(0, 0)
        for blk_i in range(blocks):
            lb = blk_i % 2
            wait_loc(lb)
            if blk_i + 1 < blocks:
                start_loc(blk_i + 1, 1 - lb)
            bt = wid * blocks + blk_i

            if blk_i == 0:
                @pl.loop(0, n_h // NBUF)
                def _(i, bt=bt, lb=lb):
                    h0 = i * NBUF
                    pl.when(i > 0)(lambda: wait_out(0))
                    process_h(h0, bt, 0, lb)
                    pl.when(i > 0)(lambda: wait_out(1))
                    process_h(h0 + 1, bt, 1, lb)
            else:
                @pl.loop(0, n_h // NBUF)
                def _(i, bt=bt, lb=lb):
                    h0 = i * NBUF
                    wait_out(0)
                    process_h(h0, bt, 0, lb)
                    wait_out(1)
                    process_h(h0 + 1, bt, 1, lb)

        wait_out(0)
        wait_out(1)

    return k


def kernel(location, table):
    b, h = location.shape
    q = _make_lookup(b, h)(table.T, location.astype(jnp.int32))
    q5 = q.reshape(h, EMBED // 8, b // BLK_B, 8, BLK_B)
    return q5.transpose(2, 4, 0, 1, 3).reshape(b, h, EMBED)


# speedup vs baseline: 1.0774x; 1.0110x over previous
import functools

import jax
import jax.numpy as jnp
from jax import lax
from jax.experimental import pallas as pl
from jax.experimental.pallas import tpu as pltpu
from jax.experimental.pallas import tpu_sc as plsc

VOCAB_ROWS = 202
EMBED = 32
BLK_B = 128
NBUF = 4
TILE = 8 * BLK_B


def _make_lookup(n_b, n_h):
    info = plsc.get_sparse_core_info()
    nc, ns = info.num_cores, info.num_subcores
    nw = nc * ns
    b_per_w = n_b // nw
    blocks = b_per_w // BLK_B
    dt_n = EMBED // 8
    n_bt = n_b // BLK_B

    mesh = plsc.VectorSubcoreMesh(core_axis_name="c", subcore_axis_name="s")

    @functools.partial(
        pl.kernel,
        mesh=mesh,
        compiler_params=pltpu.CompilerParams(
            use_tc_tiling_on_sc=False, needs_layout_passes=False
        ),
        out_type=jax.ShapeDtypeStruct((n_h, dt_n, n_bt * TILE), jnp.float32),
        scratch_types=[
            pltpu.VMEM((EMBED, VOCAB_ROWS), jnp.float32),
            pltpu.VMEM((2, BLK_B, n_h), jnp.int32),
            pltpu.VMEM((NBUF, dt_n * TILE), jnp.float32),
            pltpu.SemaphoreType.DMA((NBUF,)),
            pltpu.SemaphoreType.DMA((2,)),
        ],
    )
    def k(table_hbm, loc_hbm, out_hbm, table_v, loc_v, out_v, sem_out,
          sem_loc):
        wid = lax.axis_index("s") * nc + lax.axis_index("c")
        pltpu.sync_copy(table_hbm, table_v)
        iota16 = lax.iota(jnp.int32, 16)

        def start_loc(blk_i, lb):
            bt = wid * blocks + blk_i
            pltpu.async_copy(
                loc_hbm.at[pl.ds(bt * BLK_B, BLK_B), :], loc_v.at[lb],
                sem_loc.at[lb],
            )

        def wait_loc(lb):
            pltpu.make_async_copy(
                loc_hbm.at[pl.ds(0, BLK_B), :], loc_v.at[lb], sem_loc.at[lb]
            ).wait()

        def wait_out(buf):
            pltpu.make_async_copy(
                out_v.at[buf], out_hbm.at[0, 0, pl.ds(0, dt_n * TILE)],
                sem_out.at[buf],
            ).wait()

        def process_h(h, bt, buf, lb):
            loc_ref = loc_v.at[lb]
            idxs = [
                plsc.load_gather(
                    loc_ref,
                    [iota16 + (j * 16), jnp.full((16,), h, jnp.int32)],
                )
                for j in range(BLK_B // 16)
            ]

            @plsc.parallel_loop(0, EMBED, unroll=4)
            def _(d):
                dsplat = jnp.full((16,), 1, jnp.int32) * d
                for j in range(BLK_B // 16):
                    v = plsc.load_gather(table_v, [dsplat, idxs[j]])
                    out_v[buf, pl.ds(d * BLK_B + j * 16, 16)] = v

            for dt in range(dt_n):
                pltpu.async_copy(
                    out_v.at[buf, pl.ds(dt * TILE, TILE)],
                    out_hbm.at[h, dt, pl.ds(bt * TILE, TILE)],
                    sem_out.at[buf],
                )

        start_loc(0, 0)
        for blk_i in range(blocks):
            lb = blk_i % 2
            wait_loc(lb)
            if blk_i + 1 < blocks:
                start_loc(blk_i + 1, 1 - lb)
            bt = wid * blocks + blk_i

            if blk_i == 0:
                @pl.loop(0, n_h // NBUF)
                def _(i, bt=bt, lb=lb):
                    h0 = i * NBUF
                    for buf in range(NBUF):
                        pl.when(i > 0)(functools.partial(wait_out, buf))
                        process_h(h0 + buf, bt, buf, lb)
            else:
                @pl.loop(0, n_h // NBUF)
                def _(i, bt=bt, lb=lb):
                    h0 = i * NBUF
                    for buf in range(NBUF):
                        wait_out(buf)
                        process_h(h0 + buf, bt, buf, lb)

        for buf in range(NBUF):
            wait_out(buf)

    return k


def kernel(location, table):
    b, h = location.shape
    q = _make_lookup(b, h)(table.T, location.astype(jnp.int32))
    q5 = q.reshape(h, EMBED // 8, b // BLK_B, 8, BLK_B)
    return q5.transpose(2, 4, 0, 1, 3).reshape(b, h, EMBED)
